# Initial kernel scaffold; baseline (speedup 1.0000x reference)
import jax
import jax.numpy as jnp
from jax.experimental import pallas as pl


def kernel(keys, table):
    raise NotImplementedError("write your pallas kernel here")


# speedup vs baseline: 2.5732x; 2.5732x over previous
import functools

import jax
import jax.numpy as jnp
from jax import lax
from jax.experimental import pallas as pl
from jax.experimental.pallas import tpu as pltpu
from jax.experimental.pallas import tpu_sc as plsc

B = 16384
H = 50
D = 64
NC = 2
NS = 16
NW = NC * NS
BPW = B // NW
CB = 2
IDXC = CB * H
NCH = BPW // CB
LANES = 16
DV = D // LANES

_mesh = plsc.VectorSubcoreMesh(core_axis_name="c", subcore_axis_name="s")


@functools.partial(
    pl.kernel,
    mesh=_mesh,
    out_type=jax.ShapeDtypeStruct((B, D), jnp.float32),
    scratch_types=[
        pltpu.VMEM((NCH, IDXC), jnp.int32),
        pltpu.VMEM((IDXC, D), jnp.float32),
        pltpu.VMEM((IDXC, D), jnp.float32),
        pltpu.VMEM((BPW, D), jnp.float32),
        pltpu.SemaphoreType.DMA,
        pltpu.SemaphoreType.DMA,
    ],
    compiler_params=pltpu.CompilerParams(use_tc_tiling_on_sc=False),
)
def _sc_kernel(keys_hbm, table_hbm, out_hbm, idx_v, rows0, rows1, out_v,
               sem0, sem1):
    wid = lax.axis_index("s") * NC + lax.axis_index("c")

    pltpu.sync_copy(keys_hbm.at[wid], idx_v)

    rows = (rows0, rows1)
    sems = (sem0, sem1)

    pltpu.async_copy(table_hbm.at[idx_v.at[0]], rows0, sem0)
    pltpu.async_copy(table_hbm.at[idx_v.at[1]], rows1, sem1)

    def step(i, _):
        for b in range(2):
            jj = i * 2 + b
            rbuf, sem = rows[b], sems[b]
            pltpu.make_async_copy(table_hbm.at[idx_v.at[jj]], rbuf, sem).wait()
            for r in range(CB):
                for d in range(DV):
                    acc = jnp.zeros((LANES,), jnp.float32)
                    for l in range(H):
                        acc = jnp.maximum(
                            acc, rbuf[r * H + l, pl.ds(d * LANES, LANES)])
                    out_v[jj * CB + r, pl.ds(d * LANES, LANES)] = acc
            @pl.when(jj + 2 < NCH)
            def _():
                pltpu.async_copy(table_hbm.at[idx_v.at[jj + 2]], rbuf, sem)
        return ()

    lax.fori_loop(0, NCH // 2, step, (), unroll=False)

    pltpu.sync_copy(out_v, out_hbm.at[pl.ds(wid * BPW, BPW)])


def kernel(keys, table):
    keys3 = keys.reshape(NW, NCH, IDXC)
    return _sc_kernel(keys3, table)


# speedup vs baseline: 5.0754x; 1.9724x over previous
import functools

import jax
import jax.numpy as jnp
from jax import lax
from jax.experimental import pallas as pl
from jax.experimental.pallas import tpu as pltpu
from jax.experimental.pallas import tpu_sc as plsc

B = 16384
H = 50
D = 64
NUM_KEYS = 1000000
NUM_KEYS_PAD = 1007616
NC = 2
NS = 16
NW = NC * NS
BPW = B // NW
CB = 2
IDXC = CB * H
NCH = BPW // CB
LANES = 16
DV = D // LANES

_mesh = plsc.VectorSubcoreMesh(core_axis_name="c", subcore_axis_name="s")

TK = 8192
TGRID = NUM_KEYS_PAD // TK
TSH = (TK // 2).bit_length() - 1


def _transpose_body(tt_ref, out_ref):
    w = jnp.concatenate([tt_ref[:, 0:TK // 2], tt_ref[:, TK // 2:TK]], axis=0)
    out_ref[...] = w.T


def _tc_transpose(table_t):
    return pl.pallas_call(
        _transpose_body,
        grid=(TGRID,),
        in_specs=[pl.BlockSpec((D, TK), lambda j: (0, j))],
        out_specs=pl.BlockSpec((TK // 2, 2 * D), lambda j: (j, 0)),
        out_shape=jax.ShapeDtypeStruct((NUM_KEYS_PAD // 2, 2 * D),
                                       jnp.float32),
    )(table_t)


@functools.partial(
    pl.kernel,
    mesh=_mesh,
    out_type=jax.ShapeDtypeStruct((B, D), jnp.float32),
    scratch_types=[
        pltpu.VMEM((NCH, IDXC), jnp.int32),
        pltpu.VMEM((IDXC, D), jnp.float32),
        pltpu.VMEM((IDXC, D), jnp.float32),
        pltpu.VMEM((BPW, D), jnp.float32),
        pltpu.SemaphoreType.DMA,
        pltpu.SemaphoreType.DMA,
    ],
    compiler_params=pltpu.CompilerParams(use_tc_tiling_on_sc=False),
)
def _sc_kernel(keys_hbm, table_hbm, out_hbm, idx_v, rows0, rows1, out_v,
               sem0, sem1):
    wid = lax.axis_index("s") * NC + lax.axis_index("c")

    pltpu.sync_copy(keys_hbm.at[wid], idx_v)

    rows = (rows0, rows1)
    sems = (sem0, sem1)

    pltpu.async_copy(table_hbm.at[idx_v.at[0]], rows0, sem0)
    pltpu.async_copy(table_hbm.at[idx_v.at[1]], rows1, sem1)

    def step(i, _):
        for b in range(2):
            jj = i * 2 + b
            rbuf, sem = rows[b], sems[b]
            pltpu.make_async_copy(table_hbm.at[idx_v.at[jj]], rbuf, sem).wait()
            for r in range(CB):
                for d in range(DV):
                    acc = jnp.zeros((LANES,), jnp.float32)
                    for l in range(H):
                        acc = jnp.maximum(
                            acc, rbuf[r * H + l, pl.ds(d * LANES, LANES)])
                    out_v[jj * CB + r, pl.ds(d * LANES, LANES)] = acc
            @pl.when(jj + 2 < NCH)
            def _():
                pltpu.async_copy(table_hbm.at[idx_v.at[jj + 2]], rbuf, sem)
        return ()

    lax.fori_loop(0, NCH // 2, step, (), unroll=False)

    pltpu.sync_copy(out_v, out_hbm.at[pl.ds(wid * BPW, BPW)])


def kernel(keys, table):
    table_rm = _tc_transpose(table.T).reshape(NUM_KEYS_PAD, D)
    kk = (keys & ~(TK - 1)) + 2 * (keys & (TK // 2 - 1)) + ((keys >> TSH) & 1)
    keys3 = kk.reshape(NW, NCH, IDXC)
    return _sc_kernel(keys3, table_rm)
